# Initial kernel scaffold; baseline (speedup 1.0000x reference)
import functools
import math

import jax
import jax.numpy as jnp
from jax import lax
from jax.experimental import pallas as pl
from jax.experimental.pallas import tpu as pltpu
from jax.experimental.pallas import tpu_sc as plsc

D = 128
NC = 2
NS = 16
CHUNK = 128
ROWS_PER_TILE = 640
ACC_ROWS = NS * ROWS_PER_TILE


def _chunks_per_tile(e: int) -> int:
    return math.ceil(e / (NS * CHUNK))


def _zero_accum(s, accum, gbuf):
    zero = jnp.zeros((16,), jnp.float32)

    def zrow(i, carry):
        for j in range(D // 16):
            gbuf[i, pl.ds(j * 16, 16)] = zero
        return carry

    lax.fori_loop(0, CHUNK, zrow, 0)
    base = s * ROWS_PER_TILE
    for k in range(ROWS_PER_TILE // CHUNK):
        pltpu.sync_copy(gbuf, accum.at[pl.ds(base + k * CHUNK, CHUNK)])


def _process_edges(s, nch, x_hbm, row2d, col2d, accum, cidx, ridx, gbuf, sem):
    pltpu.sync_copy(col2d.at[pl.ds(s * nch, nch)], cidx.at[pl.ds(0, nch)])
    pltpu.sync_copy(row2d.at[pl.ds(s * nch, nch)], ridx.at[pl.ds(0, nch)])

    def chunk_body(k, carry):
        pltpu.async_copy(x_hbm.at[cidx.at[k]], gbuf, sem).wait()
        pltpu.sync_copy(gbuf, accum.at[ridx.at[k]], add=True)
        return carry

    lax.fori_loop(0, nch, chunk_body, 0)


def _write_out(s, accum, out_hbm, col0):
    base = s * ROWS_PER_TILE

    @pl.when(s < NS - 1)
    def _():
        pltpu.sync_copy(
            accum.at[pl.ds(base, ROWS_PER_TILE)],
            out_hbm.at[pl.ds(base, ROWS_PER_TILE), pl.ds(col0, D)],
        )

    @pl.when(s == NS - 1)
    def _():
        last = 10000 - (NS - 1) * ROWS_PER_TILE
        pltpu.sync_copy(
            accum.at[pl.ds(base, last)],
            out_hbm.at[pl.ds(base, last), pl.ds(col0, D)],
        )


def _make_sc_spmm(n_nodes, nch1, nch2):
    mesh = plsc.VectorSubcoreMesh(core_axis_name="c", subcore_axis_name="s")
    nch_max = max(nch1, nch2)

    @functools.partial(
        pl.kernel,
        out_type=jax.ShapeDtypeStruct((n_nodes, 2 * D), jnp.float32),
        mesh=mesh,
        scratch_types=[
            pltpu.VMEM_SHARED((ACC_ROWS, D), jnp.float32),
            pltpu.VMEM((nch_max, CHUNK), jnp.int32),
            pltpu.VMEM((nch_max, CHUNK), jnp.int32),
            pltpu.VMEM((CHUNK, D), jnp.float32),
            pltpu.SemaphoreType.DMA,
        ],
    )
    def spmm_kernel(x_hbm, row1, col1, row2, col2, out_hbm,
                    accum, cidx, ridx, gbuf, sem):
        c = lax.axis_index("c")
        s = lax.axis_index("s")

        _zero_accum(s, accum, gbuf)
        plsc.subcore_barrier()

        @pl.when(c == 0)
        def _():
            _process_edges(s, nch1, x_hbm, row1, col1, accum, cidx, ridx,
                           gbuf, sem)

        @pl.when(c == 1)
        def _():
            _process_edges(s, nch2, x_hbm, row2, col2, accum, cidx, ridx,
                           gbuf, sem)

        plsc.subcore_barrier()

        @pl.when(c == 0)
        def _():
            _write_out(s, accum, out_hbm, 0)

        @pl.when(c == 1)
        def _():
            _write_out(s, accum, out_hbm, D)

    return spmm_kernel


def _prep_edges(edge_index, n_nodes):
    e = edge_index.shape[1]
    nch = _chunks_per_tile(e)
    ep = nch * NS * CHUNK
    row = edge_index[0].astype(jnp.int32)
    col = edge_index[1].astype(jnp.int32)
    row = jnp.pad(row, (0, ep - e), constant_values=ACC_ROWS - 1)
    col = jnp.pad(col, (0, ep - e), constant_values=0)
    return row.reshape(-1, CHUNK), col.reshape(-1, CHUNK), nch


def kernel(x, edge_index, edge_index2):
    n_nodes = x.shape[0]
    row1, col1, nch1 = _prep_edges(edge_index, n_nodes)
    row2, col2, nch2 = _prep_edges(edge_index2, n_nodes)
    spmm = _make_sc_spmm(n_nodes, nch1, nch2)
    return spmm(x, row1, col1, row2, col2)


# speedup vs baseline: 3.7417x; 3.7417x over previous
import functools
import math

import jax
import jax.numpy as jnp
from jax import lax
from jax.experimental import pallas as pl
from jax.experimental.pallas import tpu as pltpu
from jax.experimental.pallas import tpu_sc as plsc

D = 128
NC = 2
NS = 16
CHUNK = 128
ROWS_PER_TILE = 640
ACC_ROWS = NS * ROWS_PER_TILE


def _chunks_per_tile(e: int) -> int:
    return 8 * math.ceil(e / (NS * CHUNK * 8))


def _zero_accum(s, accum, gbuf):
    zero = jnp.zeros((16,), jnp.float32)

    def zrow(i, carry):
        for j in range(D // 16):
            gbuf[i, pl.ds(j * 16, 16)] = zero
        return carry

    lax.fori_loop(0, CHUNK, zrow, 0)
    base = s * ROWS_PER_TILE
    for k in range(ROWS_PER_TILE // CHUNK):
        pltpu.sync_copy(gbuf, accum.at[pl.ds(base + k * CHUNK, CHUNK)])


IDX_BATCH = 8


def _process_edges(s, nch, x_hbm, row2d, col2d, accum, cidx, ridx, gbuf, sem):
    tile_base = s * nch

    def group_body(g, carry):
        pltpu.sync_copy(col2d.at[pl.ds(tile_base + g * IDX_BATCH, IDX_BATCH)],
                        cidx)
        pltpu.sync_copy(row2d.at[pl.ds(tile_base + g * IDX_BATCH, IDX_BATCH)],
                        ridx)

        def chunk_body(k, carry2):
            pltpu.async_copy(x_hbm.at[cidx.at[k]], gbuf, sem).wait()
            pltpu.sync_copy(gbuf, accum.at[ridx.at[k]], add=True)
            return carry2

        lax.fori_loop(0, IDX_BATCH, chunk_body, 0)
        return carry

    lax.fori_loop(0, nch // IDX_BATCH, group_body, 0)


def _write_out(s, accum, out_hbm, col0):
    base = s * ROWS_PER_TILE

    @pl.when(s < NS - 1)
    def _():
        pltpu.sync_copy(
            accum.at[pl.ds(base, ROWS_PER_TILE)],
            out_hbm.at[pl.ds(base, ROWS_PER_TILE), pl.ds(col0, D)],
        )

    @pl.when(s == NS - 1)
    def _():
        last = 10000 - (NS - 1) * ROWS_PER_TILE
        pltpu.sync_copy(
            accum.at[pl.ds(base, last)],
            out_hbm.at[pl.ds(base, last), pl.ds(col0, D)],
        )


def _make_sc_spmm(n_nodes, nch1, nch2):
    mesh = plsc.VectorSubcoreMesh(core_axis_name="c", subcore_axis_name="s")

    @functools.partial(
        pl.kernel,
        out_type=jax.ShapeDtypeStruct((n_nodes, 2 * D), jnp.float32),
        mesh=mesh,
        scratch_types=[
            pltpu.VMEM_SHARED((ACC_ROWS, D), jnp.float32),
            pltpu.VMEM((IDX_BATCH, CHUNK), jnp.int32),
            pltpu.VMEM((IDX_BATCH, CHUNK), jnp.int32),
            pltpu.VMEM((CHUNK, D), jnp.float32),
            pltpu.SemaphoreType.DMA,
        ],
    )
    def spmm_kernel(x_hbm, row1, col1, row2, col2, out_hbm,
                    accum, cidx, ridx, gbuf, sem):
        c = lax.axis_index("c")
        s = lax.axis_index("s")

        _zero_accum(s, accum, gbuf)
        plsc.subcore_barrier()

        @pl.when(c == 0)
        def _():
            _process_edges(s, nch1, x_hbm, row1, col1, accum, cidx, ridx,
                           gbuf, sem)

        @pl.when(c == 1)
        def _():
            _process_edges(s, nch2, x_hbm, row2, col2, accum, cidx, ridx,
                           gbuf, sem)

        plsc.subcore_barrier()

        @pl.when(c == 0)
        def _():
            _write_out(s, accum, out_hbm, 0)

        @pl.when(c == 1)
        def _():
            _write_out(s, accum, out_hbm, D)

    return spmm_kernel


def _prep_edges(edge_index, n_nodes):
    e = edge_index.shape[1]
    nch = _chunks_per_tile(e)
    ep = nch * NS * CHUNK
    row = edge_index[0].astype(jnp.int32)
    col = edge_index[1].astype(jnp.int32)
    row = jnp.pad(row, (0, ep - e), constant_values=ACC_ROWS - 1)
    col = jnp.pad(col, (0, ep - e), constant_values=0)
    return row.reshape(-1, CHUNK), col.reshape(-1, CHUNK), nch


def kernel(x, edge_index, edge_index2):
    n_nodes = x.shape[0]
    row1, col1, nch1 = _prep_edges(edge_index, n_nodes)
    row2, col2, nch2 = _prep_edges(edge_index2, n_nodes)
    spmm = _make_sc_spmm(n_nodes, nch1, nch2)
    return spmm(x, row1, col1, row2, col2)


# speedup vs baseline: 4.4172x; 1.1805x over previous
import functools
import math

import jax
import jax.numpy as jnp
from jax import lax
from jax.experimental import pallas as pl
from jax.experimental.pallas import tpu as pltpu
from jax.experimental.pallas import tpu_sc as plsc

D = 128
NC = 2
NS = 16
CHUNK = 128
G = 32
ROWS_PER_TILE = 632
ACC_ROWS = NS * ROWS_PER_TILE


def _chunks_per_tile(e: int) -> int:
    return G * math.ceil(e / (NS * CHUNK * G))


def _zero_accum(s, accum, gbuf):
    zero = jnp.zeros((16,), jnp.float32)

    def zrow(i, carry):
        for j in range(D // 16):
            gbuf[i, pl.ds(j * 16, 16)] = zero
        return carry

    lax.fori_loop(0, CHUNK, zrow, 0)
    base = s * ROWS_PER_TILE
    off = 0
    while off < ROWS_PER_TILE:
        n = min(CHUNK, ROWS_PER_TILE - off)
        pltpu.sync_copy(gbuf.at[pl.ds(0, n)], accum.at[pl.ds(base + off, n)])
        off += n


def _process_edges(s, nch, x_hbm, row2d, col2d, accum,
                   cidx, ridx, gbuf0, gbuf1, gsem0, gsem1):
    tile_base = s * nch
    gbufs = (gbuf0, gbuf1)
    gsems = (gsem0, gsem1)

    def start_gather(kk, b):
        pltpu.async_copy(x_hbm.at[cidx.at[kk]], gbufs[b], gsems[b])

    def wait_gather(b):
        pltpu.make_async_copy(x_hbm.at[cidx.at[0]], gbufs[b], gsems[b]).wait()

    def group_body(g, carry):
        base = tile_base + g * G
        pltpu.sync_copy(col2d.at[pl.ds(base, G)], cidx)
        pltpu.sync_copy(row2d.at[pl.ds(base, G)], ridx)
        start_gather(0, 0)
        for kk in range(G):
            b = kk % 2
            if kk + 1 < G:
                start_gather(kk + 1, 1 - b)
            wait_gather(b)
            pltpu.sync_copy(gbufs[b], accum.at[ridx.at[kk]], add=True)
        return carry

    lax.fori_loop(0, nch // G, group_body, 0)


def _write_out(s, n_nodes, accum, out_hbm, col0):
    base = s * ROWS_PER_TILE
    full_tiles = n_nodes // ROWS_PER_TILE
    rem = n_nodes - full_tiles * ROWS_PER_TILE

    @pl.when(s < full_tiles)
    def _():
        pltpu.sync_copy(
            accum.at[pl.ds(base, ROWS_PER_TILE)],
            out_hbm.at[pl.ds(base, ROWS_PER_TILE), pl.ds(col0, D)],
        )

    if rem > 0:
        @pl.when(s == full_tiles)
        def _():
            pltpu.sync_copy(
                accum.at[pl.ds(base, rem)],
                out_hbm.at[pl.ds(base, rem), pl.ds(col0, D)],
            )


def _make_sc_spmm(n_nodes, nch1, nch2):
    mesh = plsc.VectorSubcoreMesh(core_axis_name="c", subcore_axis_name="s")

    @functools.partial(
        pl.kernel,
        out_type=jax.ShapeDtypeStruct((n_nodes, 2 * D), jnp.float32),
        mesh=mesh,
        scratch_types=[
            pltpu.VMEM_SHARED((ACC_ROWS, D), jnp.float32),
            pltpu.VMEM((G, CHUNK), jnp.int32),
            pltpu.VMEM((G, CHUNK), jnp.int32),
            pltpu.VMEM((CHUNK, D), jnp.float32),
            pltpu.VMEM((CHUNK, D), jnp.float32),
            pltpu.SemaphoreType.DMA,
            pltpu.SemaphoreType.DMA,
        ],
    )
    def spmm_kernel(x_hbm, row1, col1, row2, col2, out_hbm,
                    accum, cidx, ridx, gbuf0, gbuf1, gsem0, gsem1):
        c = lax.axis_index("c")
        s = lax.axis_index("s")

        _zero_accum(s, accum, gbuf0)
        plsc.subcore_barrier()

        @pl.when(c == 0)
        def _():
            _process_edges(s, nch1, x_hbm, row1, col1, accum,
                           cidx, ridx, gbuf0, gbuf1, gsem0, gsem1)

        @pl.when(c == 1)
        def _():
            _process_edges(s, nch2, x_hbm, row2, col2, accum,
                           cidx, ridx, gbuf0, gbuf1, gsem0, gsem1)

        plsc.subcore_barrier()

        @pl.when(c == 0)
        def _():
            _write_out(s, n_nodes, accum, out_hbm, 0)

        @pl.when(c == 1)
        def _():
            _write_out(s, n_nodes, accum, out_hbm, D)

    return spmm_kernel


def _prep_edges(edge_index, n_nodes):
    e = edge_index.shape[1]
    nch = _chunks_per_tile(e)
    ep = nch * NS * CHUNK
    row = edge_index[0].astype(jnp.int32)
    col = edge_index[1].astype(jnp.int32)
    row = jnp.pad(row, (0, ep - e), constant_values=ACC_ROWS - 1)
    col = jnp.pad(col, (0, ep - e), constant_values=0)
    return row.reshape(-1, CHUNK), col.reshape(-1, CHUNK), nch


def kernel(x, edge_index, edge_index2):
    n_nodes = x.shape[0]
    row1, col1, nch1 = _prep_edges(edge_index, n_nodes)
    row2, col2, nch2 = _prep_edges(edge_index2, n_nodes)
    spmm = _make_sc_spmm(n_nodes, nch1, nch2)
    return spmm(x, row1, col1, row2, col2)


# speedup vs baseline: 6.4629x; 1.4631x over previous
import functools
import math

import jax
import jax.numpy as jnp
from jax import lax
from jax.experimental import pallas as pl
from jax.experimental.pallas import tpu as pltpu
from jax.experimental.pallas import tpu_sc as plsc

D = 128
DH = D // 2
NC = 2
NS = 16
CHUNK = 128
G = 32
NBUF = 4
LAG = 2
SCATTER_BYTES = CHUNK * DH * 4


def _chunks_per_tile(e: int) -> int:
    return G * math.ceil(e / (NS * CHUNK * G))


def _zero_accum(s, rows_per_tile, half, accum, gbuf):
    zero = jnp.zeros((16,), jnp.float32)

    def zrow(i, carry):
        for j in range(DH // 16):
            gbuf[i, pl.ds(j * 16, 16)] = zero
        return carry

    lax.fori_loop(0, CHUNK, zrow, 0)
    for h in range(2):
        base = h * half + s * rows_per_tile
        off = 0
        while off < rows_per_tile:
            n = min(CHUNK, rows_per_tile - off)
            pltpu.sync_copy(gbuf.at[pl.ds(0, n)],
                            accum.at[pl.ds(base + off, n)])
            off += n


def _process_edges(s, nch, x_half, row2d, col2d, accum,
                   cidx, ridx, gbufs, gsems, ssems):
    tile_base = s * nch

    def start_gather(kk, b):
        pltpu.async_copy(x_half.at[cidx.at[kk]], gbufs[b], gsems[b])

    def wait_gather(b):
        pltpu.make_async_copy(x_half.at[cidx.at[0]], gbufs[b],
                              gsems[b]).wait()

    def start_scatter(kk, b):
        pltpu.async_copy(gbufs[b], accum.at[ridx.at[kk]], ssems[b],
                         add=True)

    def wait_scatter(b):
        pltpu.make_async_copy(gbufs[b], accum.at[ridx.at[0]],
                              ssems[b]).wait()

    def group_body(g, carry):
        base = tile_base + g * G
        pltpu.sync_copy(col2d.at[pl.ds(base, G)], cidx)
        pltpu.sync_copy(row2d.at[pl.ds(base, G)], ridx)
        for kk in range(G):
            if kk >= NBUF:
                wait_scatter(kk % NBUF)
            start_gather(kk, kk % NBUF)
            j = kk - LAG
            if j >= 0:
                wait_gather(j % NBUF)
                start_scatter(j, j % NBUF)
        for j in range(G - LAG, G):
            wait_gather(j % NBUF)
            start_scatter(j, j % NBUF)
        for b in range(NBUF):
            wait_scatter(b)
        return carry

    lax.fori_loop(0, nch // G, group_body, 0)


def _write_out(s, n_nodes, rows_per_tile, half, accum, out_hbm, c):
    full_tiles = n_nodes // rows_per_tile
    rem = n_nodes - full_tiles * rows_per_tile

    def copies(cq):
        for h in range(2):
            acc_base = h * half + s * rows_per_tile
            out_base = s * rows_per_tile
            q = h * 2 + cq

            @pl.when(s < full_tiles)
            def _():
                pltpu.sync_copy(
                    accum.at[pl.ds(acc_base, rows_per_tile)],
                    out_hbm.at[q, pl.ds(out_base, rows_per_tile)],
                )

            if rem > 0:
                @pl.when(s == full_tiles)
                def _():
                    pltpu.sync_copy(
                        accum.at[pl.ds(acc_base, rem)],
                        out_hbm.at[q, pl.ds(out_base, rem)],
                    )

    @pl.when(c == 0)
    def _():
        copies(0)

    @pl.when(c == 1)
    def _():
        copies(1)


def _make_sc_spmm(n_nodes, nch):
    mesh = plsc.VectorSubcoreMesh(core_axis_name="c", subcore_axis_name="s")
    rows_per_tile = 8 * math.ceil(n_nodes / (NS * 8))
    half = NS * rows_per_tile

    @functools.partial(
        pl.kernel,
        out_type=jax.ShapeDtypeStruct((4, n_nodes, DH), jnp.float32),
        mesh=mesh,
        scratch_types=[
            pltpu.VMEM_SHARED((2 * half, DH), jnp.float32),
            pltpu.VMEM((G, CHUNK), jnp.int32),
            pltpu.VMEM((G, CHUNK), jnp.int32),
        ] + [pltpu.VMEM((CHUNK, DH), jnp.float32) for _ in range(NBUF)]
          + [pltpu.SemaphoreType.DMA for _ in range(2 * NBUF)],
        compiler_params=pltpu.CompilerParams(use_tc_tiling_on_sc=False),
    )
    def spmm_kernel(x3_hbm, row2d, col2d, out_hbm, accum, cidx, ridx, *rest):
        gbufs = rest[:NBUF]
        gsems = rest[NBUF:2 * NBUF]
        ssems = rest[2 * NBUF:]
        c = lax.axis_index("c")
        s = lax.axis_index("s")

        _zero_accum(s, rows_per_tile, half, accum, gbufs[0])
        plsc.subcore_barrier()

        _process_edges(s, nch, x3_hbm.at[c], row2d, col2d, accum,
                       cidx, ridx, gbufs, gsems, ssems)

        plsc.subcore_barrier()
        _write_out(s, n_nodes, rows_per_tile, half, accum, out_hbm, c)

    return spmm_kernel, half


def _prep_edges(edge_index, row_offset, dummy_row):
    e = edge_index.shape[1]
    nch = _chunks_per_tile(e)
    ep = nch * NS * CHUNK
    row = edge_index[0].astype(jnp.int32) + row_offset
    col = edge_index[1].astype(jnp.int32)
    row = jnp.pad(row, (0, ep - e), constant_values=dummy_row)
    col = jnp.pad(col, (0, ep - e), constant_values=0)
    return (row.reshape(NS, nch, CHUNK), col.reshape(NS, nch, CHUNK), nch)


def kernel(x, edge_index, edge_index2):
    n_nodes = x.shape[0]
    rows_per_tile = 8 * math.ceil(n_nodes / (NS * 8))
    half = NS * rows_per_tile
    r1, c1, nch1 = _prep_edges(edge_index, 0, half - 1)
    r2, c2, nch2 = _prep_edges(edge_index2, half, 2 * half - 1)
    row2d = jnp.concatenate([r1, r2], axis=1).reshape(-1, CHUNK)
    col2d = jnp.concatenate([c1, c2], axis=1).reshape(-1, CHUNK)
    x3 = jnp.stack([x[:, :DH], x[:, DH:]])
    spmm, _ = _make_sc_spmm(n_nodes, nch1 + nch2)
    out4 = spmm(x3, row2d, col2d)
    return jnp.concatenate([out4[0], out4[1], out4[2], out4[3]], axis=1)
